# baseline (speedup 1.0000x reference)
import functools

import jax
import jax.numpy as jnp
from jax import lax
from jax.experimental import pallas as pl
from jax.experimental.pallas import tpu as pltpu
from jax.experimental.pallas import tpu_sc as plsc

VOCAB = 1000000
DIM = 32
BATCH = 16384
SEQ = 50

_info = plsc.get_sparse_core_info()
_NC, _NS = _info.num_cores, _info.num_subcores
_NW = _NC * _NS

_BC = BATCH // _NW
_NSTR = _BC // 128


@functools.partial(
    pl.kernel,
    mesh=plsc.VectorSubcoreMesh(core_axis_name="c", subcore_axis_name="s"),
    compiler_params=pltpu.CompilerParams(
        use_tc_tiling_on_sc=False, needs_layout_passes=False),
    out_type=jax.ShapeDtypeStruct((SEQ, DIM, BATCH), jnp.float32),
    scratch_types=[
        pltpu.VMEM((_NSTR, 128), jnp.int32),
        pltpu.VMEM((_NSTR, 128), jnp.int32),
        pltpu.VMEM((_BC, DIM), jnp.float32),
        pltpu.VMEM((_BC, DIM), jnp.float32),
        pltpu.VMEM((DIM, _BC), jnp.float32),
        pltpu.VMEM((DIM, _BC), jnp.float32),
        pltpu.SemaphoreType.DMA,
        pltpu.SemaphoreType.DMA,
        pltpu.SemaphoreType.DMA,
        pltpu.SemaphoreType.DMA,
    ],
)
def _gather_kernel(x3, tab, out, idx0, idx1, rows0, rows1, planes0, planes1,
                   g0, g1, o0, o1):
  wid = lax.axis_index("s") * _NC + lax.axis_index("c")
  b0 = wid * _BC
  iota = lax.iota(jnp.int32, 16)
  dvecs = [jnp.full((16,), d, jnp.int32) for d in range(DIM)]

  def fire(s, idx_v, rows_v, gsem):
    pltpu.sync_copy(x3.at[s, pl.ds(wid * _NSTR, _NSTR)], idx_v)
    return [
        pltpu.async_copy(tab.at[idx_v.at[j]],
                         rows_v.at[pl.ds(j * 128, 128)], gsem)
        for j in range(_NSTR)
    ]

  def transpose(rows_v, planes_v):
    @plsc.parallel_loop(0, _BC // 16, unroll=4)
    def kloop(k):
      bvec = iota + k * 16
      vals = [plsc.load_gather(rows_v, [bvec, dvecs[d]]) for d in range(DIM)]
      for d in range(DIM):
        planes_v[d, pl.ds(k * 16, 16)] = vals[d]

  def pair(g, carry):
    a = 2 * g
    b = a + 1
    ca = fire(a, idx0, rows0, g0)
    cb = fire(b, idx1, rows1, g1)

    @pl.when(g > 0)
    def _drain_prev():
      pltpu.make_async_copy(
          planes0, out.at[a - 2, :, pl.ds(b0, _BC)], o0).wait()
      pltpu.make_async_copy(
          planes1, out.at[b - 2, :, pl.ds(b0, _BC)], o1).wait()

    for c in ca:
      c.wait()
    transpose(rows0, planes0)
    pltpu.async_copy(planes0, out.at[a, :, pl.ds(b0, _BC)], o0)

    for c in cb:
      c.wait()
    transpose(rows1, planes1)
    pltpu.async_copy(planes1, out.at[b, :, pl.ds(b0, _BC)], o1)
    return carry

  lax.fori_loop(0, SEQ // 2, pair, 0)
  pltpu.make_async_copy(
      planes0, out.at[SEQ - 2, :, pl.ds(b0, _BC)], o0).wait()
  pltpu.make_async_copy(
      planes1, out.at[SEQ - 1, :, pl.ds(b0, _BC)], o1).wait()


def kernel(X, table):
  x3 = X.T.reshape(SEQ, BATCH // 128, 128)
  out = _gather_kernel(x3, table)
  return out.transpose(2, 0, 1)


# speedup vs baseline: 1.2212x; 1.2212x over previous
import functools

import jax
import jax.numpy as jnp
from jax import lax
from jax.experimental import pallas as pl
from jax.experimental.pallas import tpu as pltpu
from jax.experimental.pallas import tpu_sc as plsc

VOCAB = 1000000
DIM = 32
BATCH = 16384
SEQ = 50

_info = plsc.get_sparse_core_info()
_NC, _NS = _info.num_cores, _info.num_subcores
_NW = _NC * _NS

_BC = BATCH // _NW
_NSTR = _BC // 128


@functools.partial(
    pl.kernel,
    mesh=plsc.VectorSubcoreMesh(core_axis_name="c", subcore_axis_name="s"),
    compiler_params=pltpu.CompilerParams(
        use_tc_tiling_on_sc=False, needs_layout_passes=False),
    out_type=jax.ShapeDtypeStruct((SEQ, DIM, BATCH), jnp.float32),
    scratch_types=[
        pltpu.VMEM((_NSTR, 128), jnp.int32),
        pltpu.VMEM((_NSTR, 128), jnp.int32),
        pltpu.VMEM((_BC, DIM), jnp.float32),
        pltpu.VMEM((_BC, DIM), jnp.float32),
        pltpu.VMEM((DIM, _BC), jnp.float32),
        pltpu.VMEM((DIM, _BC), jnp.float32),
        pltpu.SemaphoreType.DMA,
        pltpu.SemaphoreType.DMA,
        pltpu.SemaphoreType.DMA,
        pltpu.SemaphoreType.DMA,
    ],
)
def _gather_kernel(x3, tab, out, idx0, idx1, rows0, rows1, planes0, planes1,
                   g0, g1, o0, o1):
  wid = lax.axis_index("s") * _NC + lax.axis_index("c")
  b0 = wid * _BC
  iota = lax.iota(jnp.int32, 16)
  dvecs = [jnp.full((16,), d, jnp.int32) for d in range(DIM)]

  def fire(s, idx_v, rows_v, gsem):
    pltpu.sync_copy(x3.at[s, pl.ds(wid * _NSTR, _NSTR)], idx_v)
    return [
        pltpu.async_copy(tab.at[idx_v.at[j]],
                         rows_v.at[pl.ds(j * 128, 128)], gsem)
        for j in range(_NSTR)
    ]

  def transpose(rows_v, planes_v):
    @plsc.parallel_loop(0, _BC // 16, unroll=2)
    def kloop(k):
      bvec = iota + k * 16
      vals = [plsc.load_gather(rows_v, [bvec, dvecs[d]]) for d in range(DIM)]
      for d in range(DIM):
        planes_v[d, pl.ds(k * 16, 16)] = vals[d]

  def pair(g, carry):
    a = 2 * g
    b = a + 1
    ca = fire(a, idx0, rows0, g0)
    cb = fire(b, idx1, rows1, g1)

    @pl.when(g > 0)
    def _drain_prev():
      pltpu.make_async_copy(
          planes0, out.at[a - 2, :, pl.ds(b0, _BC)], o0).wait()
      pltpu.make_async_copy(
          planes1, out.at[b - 2, :, pl.ds(b0, _BC)], o1).wait()

    for c in ca:
      c.wait()
    transpose(rows0, planes0)
    pltpu.async_copy(planes0, out.at[a, :, pl.ds(b0, _BC)], o0)

    for c in cb:
      c.wait()
    transpose(rows1, planes1)
    pltpu.async_copy(planes1, out.at[b, :, pl.ds(b0, _BC)], o1)
    return carry

  lax.fori_loop(0, SEQ // 2, pair, 0)
  pltpu.make_async_copy(
      planes0, out.at[SEQ - 2, :, pl.ds(b0, _BC)], o0).wait()
  pltpu.make_async_copy(
      planes1, out.at[SEQ - 1, :, pl.ds(b0, _BC)], o1).wait()


def kernel(X, table):
  x3 = X.T.reshape(SEQ, BATCH // 128, 128)
  out = _gather_kernel(x3, table)
  return out.transpose(2, 0, 1)
